# baseline (speedup 1.0000x reference)
import functools

import jax
import jax.numpy as jnp
from jax import lax
from jax.experimental import pallas as pl
from jax.experimental.pallas import tpu as pltpu
from jax.experimental.pallas import tpu_sc as plsc

_K_PERCENT = (10, 10)

_B, _C, _H, _W = 8, 2, 512, 512
_N = _B * _H * _W
_KK = int(_N * _K_PERCENT[0] / 100)
_NS = 16
_NT = _N // _NS
_CH = 16384
_NCHUNK = _NT // _CH
_BINS = 256
_L = 16
_CAPL = 1024


def _bce_body(p_ref, t_ref, out_ref):
    p = p_ref[...]
    t = t_ref[...]
    log_p = jnp.maximum(jnp.log(p), -100.0)
    log_1mp = jnp.maximum(jnp.log(1.0 - p), -100.0)
    out_ref[...] = (-(t * log_p + (1.0 - t) * log_1mp)).reshape(1, _H, _W)


def _bce_loss_channel_major(predict, target):
    return pl.pallas_call(
        _bce_body,
        grid=(_C, _B),
        in_specs=[
            pl.BlockSpec((1, 1, _H, _W), lambda c, b: (b, c, 0, 0)),
            pl.BlockSpec((1, 1, _H, _W), lambda c, b: (b, c, 0, 0)),
        ],
        out_specs=pl.BlockSpec((1, _H, _W), lambda c, b: (c, b, 0)),
        out_shape=jax.ShapeDtypeStruct((_C, _B * _H, _W), jnp.float32),
    )(predict, target)


def _sc_body(loss_ref, out_ref, buf, cnt, sm, cbuf, mbuf, macc, stage, mc,
             ms, sema, semb, sh_cnt, sh_sum, sh_mc, sh_ms):
    ch = lax.axis_index("c")
    s = lax.axis_index("s")
    lane = lax.iota(jnp.int32, _L)
    ones = jnp.ones((_L,), jnp.float32)
    zeros16 = jnp.zeros((_L,), jnp.float32)
    lane_base = lane * _CAPL

    rem = jnp.full((_L,), float(_KK), jnp.float32)
    s_inc = jnp.zeros((_L,), jnp.float32)
    pref = jnp.zeros((_L,), jnp.int32)
    sacc = jnp.zeros((_L,), jnp.float32)
    offs = jnp.zeros((_L,), jnp.int32)

    rows_per_tile = _NT // _W
    rows_per_chunk = _CH // _W

    def chunk_src(j):
        return loss_ref.at[
            ch, pl.ds(s * rows_per_tile + j * rows_per_chunk, rows_per_chunk), :]

    def merge_and_scan(with_sums, rem, s_inc):
        plsc.subcore_barrier()
        pairs = ((sh_cnt, sh_mc), (sh_sum, sh_ms)) if with_sums else (
            (sh_cnt, sh_mc),)
        for src, dst in pairs:
            pltpu.sync_copy(src.at[:, pl.ds(s * _BINS, _BINS)], mbuf)

            @plsc.parallel_loop(0, _L, 1, unroll=4)
            def zacc(q):
                macc[pl.ds(q * _L, _L)] = zeros16

            def row_body(r, _):
                @plsc.parallel_loop(0, _L, 1, unroll=4)
                def q_body(q):
                    macc[pl.ds(q * _L, _L)] = (
                        macc[pl.ds(q * _L, _L)] + mbuf[r, pl.ds(q * _L, _L)])
                return 0
            lax.fori_loop(0, _NS, row_body, 0)

            def lred(k, tot):
                return jnp.where(lane == k,
                                 jnp.sum(macc[pl.ds(k * _L, _L)]), tot)
            totv = lax.fori_loop(0, _L, lred, zeros16)
            stage[...] = totv
            pltpu.sync_copy(stage, dst.at[pl.ds(s * _L, _L)])
        plsc.subcore_barrier()

        pltpu.sync_copy(sh_mc, mc)
        if with_sums:
            pltpu.sync_copy(sh_ms, ms)

        def scan_body(gi, carry):
            if with_sums:
                c0, s1, rem1, binv, found = carry
            else:
                c0, rem1, binv, found = carry
            g = 15 - gi
            cv = mc[pl.ds(g * _L, _L)]
            rv = lax.rev(cv, (0,))
            cum = plsc.cumsum(rv)
            gtot = jnp.sum(cv)
            any_hit = (c0 + gtot) >= rem1
            hit = jnp.logical_and(jnp.logical_not(found),
                                  (c0 + cum) >= rem1)
            i = plsc.all_reduce_ffs(hit)
            sel = lane == i
            rv_i = jnp.sum(jnp.where(sel, rv, 0.0))
            cum_i = jnp.sum(jnp.where(sel, cum, 0.0))
            in_group = jnp.logical_and(jnp.logical_not(found), any_hit)
            rem2 = jnp.where(in_group, rem1 - (c0 + cum_i - rv_i), rem1)
            binv2 = jnp.where(in_group, g * _L + (15 - i), binv)
            found2 = jnp.logical_or(found, in_group)
            if with_sums:
                sv = ms[pl.ds(g * _L, _L)]
                rs = lax.rev(sv, (0,))
                cums = plsc.cumsum(rs)
                gstot = jnp.sum(sv)
                cums_i = jnp.sum(jnp.where(sel, cums, 0.0))
                rs_i = jnp.sum(jnp.where(sel, rs, 0.0))
                take_all = jnp.logical_and(jnp.logical_not(found),
                                           jnp.logical_not(any_hit))
                s2 = jnp.where(in_group, s1 + (cums_i - rs_i),
                               jnp.where(take_all, s1 + gstot, s1))
                return (c0 + gtot, s2, rem2, binv2, found2)
            return (c0 + gtot, rem2, binv2, found2)

        zi = jnp.zeros((_L,), jnp.int32)
        zb = jnp.zeros((_L,), jnp.bool_)
        if with_sums:
            init = (zeros16, s_inc, rem, zi, zb)
            _, s_inc, rem, binstar, _ = lax.fori_loop(0, _L, scan_body, init)
        else:
            init = (zeros16, rem, zi, zb)
            _, rem, binstar, _ = lax.fori_loop(0, _L, scan_body, init)
        return rem, s_inc, binstar

    for shift in (24, 16, 8):
        compact = shift == 8

        @plsc.parallel_loop(0, _BINS, 1, unroll=8)
        def zero_body(q):
            cnt[pl.ds(q * _L, _L)] = zeros16

        pref_now = pref
        pref_hi = pref_now | 255

        def process(p, acc):
            @plsc.parallel_loop(0, _CH // _L, 1, unroll=16, carry=acc)
            def it_body(i, a):
                r = lax.shift_right_logical(i, 5)
                col = (i & 31) * _L
                v = buf[p, r, pl.ds(col, _L)]
                bits = lax.bitcast_convert_type(v, jnp.int32)
                b = lax.shift_right_logical(bits, shift) & 255
                idx = b * _L + lane
                if shift == 24:
                    plsc.addupdate_scatter(cnt, [idx], ones)
                    return a
                hi = lax.shift_right_logical(bits, shift + 8)
                m = hi == pref_now
                plsc.addupdate_scatter(cnt, [idx], ones, mask=m)
                if shift == 16:
                    m_above = hi > pref_now
                else:
                    m_above = jnp.logical_and(hi > pref_now, hi <= pref_hi)
                av, off = a
                av = av + jnp.where(m_above, v, 0.0)
                if compact:
                    slot = lane_base + jnp.minimum(off, _CAPL - 1)
                    plsc.store_scatter(cbuf, [slot], v, mask=m)
                    off = off + jnp.where(m, 1, 0)
                return (av, off)
            return it_body

        pltpu.async_copy(chunk_src(0), buf.at[0], sema)
        pltpu.async_copy(chunk_src(1), buf.at[1], semb)

        def dchunk_body(m_, acc):
            j0 = 2 * m_
            pltpu.make_async_copy(chunk_src(j0), buf.at[0], sema).wait()
            acc = process(0, acc)
            pltpu.async_copy(chunk_src(j0 + 2), buf.at[0], sema)
            pltpu.make_async_copy(chunk_src(j0 + 1), buf.at[1], semb).wait()
            acc = process(1, acc)
            pltpu.async_copy(chunk_src(j0 + 3), buf.at[1], semb)
            return acc
        if shift == 24:
            sacc = lax.fori_loop(0, _NCHUNK // 2 - 1, dchunk_body, sacc)
        else:
            sacc, offs = lax.fori_loop(
                0, _NCHUNK // 2 - 1, dchunk_body, (sacc, offs))
        pltpu.make_async_copy(chunk_src(_NCHUNK - 2), buf.at[0], sema).wait()
        acc_t = sacc if shift == 24 else (sacc, offs)
        acc_t = process(0, acc_t)
        pltpu.make_async_copy(chunk_src(_NCHUNK - 1), buf.at[1], semb).wait()
        acc_t = process(1, acc_t)
        if shift == 24:
            sacc = acc_t
        else:
            sacc, offs = acc_t

        pltpu.sync_copy(cnt, sh_cnt.at[s])
        rem, s_inc, binstar = merge_and_scan(False, rem, s_inc)
        pref = lax.shift_left(pref, 8) | binstar

    @plsc.parallel_loop(0, _BINS, 1, unroll=8)
    def zero4_body(q):
        cnt[pl.ds(q * _L, _L)] = zeros16
        sm[pl.ds(q * _L, _L)] = zeros16

    pref24 = pref
    maxoff = jnp.max(offs)

    def cbody(i, a):
        slot = lane_base + i
        valid = i < offs
        v = plsc.load_gather(cbuf, [slot], mask=valid)
        bits = lax.bitcast_convert_type(v, jnp.int32)
        hi8 = lax.shift_right_logical(bits, 8)
        m = jnp.logical_and(valid, hi8 == pref24)
        b = bits & 255
        idx = b * _L + lane
        plsc.addupdate_scatter(cnt, [idx], ones, mask=m)
        plsc.addupdate_scatter(sm, [idx], v, mask=m)
        m_above = jnp.logical_and(valid, hi8 > pref24)
        return a + jnp.where(m_above, v, 0.0)
    sacc = lax.fori_loop(0, maxoff, cbody, sacc)

    pltpu.sync_copy(cnt, sh_cnt.at[s])
    pltpu.sync_copy(sm, sh_sum.at[s])
    rem, s_inc, binstar = merge_and_scan(True, rem, s_inc)
    pref = lax.shift_left(pref, 8) | binstar

    plsc.subcore_barrier()
    stage[...] = sacc
    pltpu.sync_copy(stage, sh_mc.at[pl.ds(s * _L, _L)])
    plsc.subcore_barrier()
    pltpu.sync_copy(sh_mc, mc)

    def sred(q, tot):
        return tot + mc[pl.ds(q * _L, _L)]
    sacc_rows = lax.fori_loop(0, _L, sred, zeros16)
    s_above = jnp.sum(sacc_rows)

    tval = lax.bitcast_convert_type(pref, jnp.float32)
    res = s_inc + s_above + rem * tval

    @pl.when(s == 0)
    def _():
        stage[...] = res
        pltpu.sync_copy(stage, out_ref.at[ch])


def _sc_topk_sums(loss):
    mesh = plsc.VectorSubcoreMesh(core_axis_name="c", subcore_axis_name="s")
    f = pl.kernel(
        _sc_body,
        out_type=jax.ShapeDtypeStruct((_C, _L), jnp.float32),
        mesh=mesh,
        compiler_params=pltpu.CompilerParams(
            needs_layout_passes=False, use_tc_tiling_on_sc=True),
        scratch_types=[
            pltpu.VMEM((2, _CH // _W, _W), jnp.float32),
            pltpu.VMEM((_BINS * _L,), jnp.float32),
            pltpu.VMEM((_BINS * _L,), jnp.float32),
            pltpu.VMEM((_L * _CAPL,), jnp.float32),
            pltpu.VMEM((_NS, _BINS), jnp.float32),
            pltpu.VMEM((_BINS,), jnp.float32),
            pltpu.VMEM((_L,), jnp.float32),
            pltpu.VMEM((_BINS,), jnp.float32),
            pltpu.VMEM((_BINS,), jnp.float32),
            pltpu.SemaphoreType.DMA,
            pltpu.SemaphoreType.DMA,
            pltpu.VMEM_SHARED((_NS, _BINS * _L), jnp.float32),
            pltpu.VMEM_SHARED((_NS, _BINS * _L), jnp.float32),
            pltpu.VMEM_SHARED((_BINS,), jnp.float32),
            pltpu.VMEM_SHARED((_BINS,), jnp.float32),
        ],
    )
    return f(loss)


def kernel(predict, target, is_average):
    loss = _bce_loss_channel_major(predict, target)
    sums = _sc_topk_sums(loss)
    total = (sums[0, 0] + sums[1, 0]) / (_KK * _C)
    return jnp.where(is_average, total, total * _B)


# speedup vs baseline: 1.0293x; 1.0293x over previous
import functools

import jax
import jax.numpy as jnp
from jax import lax
from jax.experimental import pallas as pl
from jax.experimental.pallas import tpu as pltpu
from jax.experimental.pallas import tpu_sc as plsc

_K_PERCENT = (10, 10)

_B, _C, _H, _W = 8, 2, 512, 512
_N = _B * _H * _W
_KK = int(_N * _K_PERCENT[0] / 100)
_NS = 16
_NT = _N // _NS
_CH = 16384
_NCHUNK = _NT // _CH
_BINS = 256
_L = 16
_CAPL = 1024


def _bce_body(p_ref, t_ref, out_ref):
    p = p_ref[...]
    t = t_ref[...]
    log_p = jnp.maximum(jnp.log(p), -100.0)
    log_1mp = jnp.maximum(jnp.log(1.0 - p), -100.0)
    out_ref[...] = (-(t * log_p + (1.0 - t) * log_1mp)).reshape(1, _H, _W)


def _bce_loss_channel_major(predict, target):
    return pl.pallas_call(
        _bce_body,
        grid=(_C, _B),
        in_specs=[
            pl.BlockSpec((1, 1, _H, _W), lambda c, b: (b, c, 0, 0)),
            pl.BlockSpec((1, 1, _H, _W), lambda c, b: (b, c, 0, 0)),
        ],
        out_specs=pl.BlockSpec((1, _H, _W), lambda c, b: (c, b, 0)),
        out_shape=jax.ShapeDtypeStruct((_C, _B * _H, _W), jnp.float32),
    )(predict, target)


def _sc_body(loss_ref, out_ref, buf, cnt, sm, cbuf, mbuf, macc, stage, mc,
             ms, sema, semb, sh_cnt, sh_sum, sh_mc, sh_ms):
    ch = lax.axis_index("c")
    s = lax.axis_index("s")
    lane = lax.iota(jnp.int32, _L)
    ones = jnp.ones((_L,), jnp.float32)
    zeros16 = jnp.zeros((_L,), jnp.float32)
    lane_base = lane * _CAPL

    rem = jnp.full((_L,), float(_KK), jnp.float32)
    s_inc = jnp.zeros((_L,), jnp.float32)
    pref = jnp.zeros((_L,), jnp.int32)
    sacc = jnp.zeros((_L,), jnp.float32)
    offs = jnp.zeros((_L,), jnp.int32)

    rows_per_tile = _NT // _W
    rows_per_chunk = _CH // _W

    def chunk_src(j):
        return loss_ref.at[
            ch, pl.ds(s * rows_per_tile + j * rows_per_chunk, rows_per_chunk), :]

    def merge_and_scan(with_sums, rem, s_inc):
        plsc.subcore_barrier()
        pairs = ((sh_cnt, sh_mc), (sh_sum, sh_ms)) if with_sums else (
            (sh_cnt, sh_mc),)
        for src, dst in pairs:
            pltpu.sync_copy(src.at[:, pl.ds(s * _BINS, _BINS)], mbuf)

            @plsc.parallel_loop(0, _L, 1, unroll=4)
            def zacc(q):
                macc[pl.ds(q * _L, _L)] = zeros16

            def row_body(r, _):
                @plsc.parallel_loop(0, _L, 1, unroll=4)
                def q_body(q):
                    macc[pl.ds(q * _L, _L)] = (
                        macc[pl.ds(q * _L, _L)] + mbuf[r, pl.ds(q * _L, _L)])
                return 0
            lax.fori_loop(0, _NS, row_body, 0)

            def lred(k, tot):
                return jnp.where(lane == k,
                                 jnp.sum(macc[pl.ds(k * _L, _L)]), tot)
            totv = lax.fori_loop(0, _L, lred, zeros16)
            stage[...] = totv
            pltpu.sync_copy(stage, dst.at[pl.ds(s * _L, _L)])
        plsc.subcore_barrier()

        pltpu.sync_copy(sh_mc, mc)
        if with_sums:
            pltpu.sync_copy(sh_ms, ms)

        def scan_body(gi, carry):
            if with_sums:
                c0, s1, rem1, binv, found = carry
            else:
                c0, rem1, binv, found = carry
            g = 15 - gi
            cv = mc[pl.ds(g * _L, _L)]
            rv = lax.rev(cv, (0,))
            cum = plsc.cumsum(rv)
            gtot = jnp.sum(cv)
            any_hit = (c0 + gtot) >= rem1
            hit = jnp.logical_and(jnp.logical_not(found),
                                  (c0 + cum) >= rem1)
            i = plsc.all_reduce_ffs(hit)
            sel = lane == i
            rv_i = jnp.sum(jnp.where(sel, rv, 0.0))
            cum_i = jnp.sum(jnp.where(sel, cum, 0.0))
            in_group = jnp.logical_and(jnp.logical_not(found), any_hit)
            rem2 = jnp.where(in_group, rem1 - (c0 + cum_i - rv_i), rem1)
            binv2 = jnp.where(in_group, g * _L + (15 - i), binv)
            found2 = jnp.logical_or(found, in_group)
            if with_sums:
                sv = ms[pl.ds(g * _L, _L)]
                rs = lax.rev(sv, (0,))
                cums = plsc.cumsum(rs)
                gstot = jnp.sum(sv)
                cums_i = jnp.sum(jnp.where(sel, cums, 0.0))
                rs_i = jnp.sum(jnp.where(sel, rs, 0.0))
                take_all = jnp.logical_and(jnp.logical_not(found),
                                           jnp.logical_not(any_hit))
                s2 = jnp.where(in_group, s1 + (cums_i - rs_i),
                               jnp.where(take_all, s1 + gstot, s1))
                return (c0 + gtot, s2, rem2, binv2, found2)
            return (c0 + gtot, rem2, binv2, found2)

        zi = jnp.zeros((_L,), jnp.int32)
        zb = jnp.zeros((_L,), jnp.bool_)
        if with_sums:
            init = (zeros16, s_inc, rem, zi, zb)
            _, s_inc, rem, binstar, _ = lax.fori_loop(0, _L, scan_body, init)
        else:
            init = (zeros16, rem, zi, zb)
            _, rem, binstar, _ = lax.fori_loop(0, _L, scan_body, init)
        return rem, s_inc, binstar

    for shift in (24, 16, 8):
        compact = shift == 8

        @plsc.parallel_loop(0, _BINS, 1, unroll=8)
        def zero_body(q):
            cnt[pl.ds(q * _L, _L)] = zeros16

        pref_now = pref
        pref_hi = pref_now | 255

        def process(p, acc):
            @plsc.parallel_loop(0, _CH // _L, 1, unroll=8, carry=acc)
            def it_body(i, a):
                r = lax.shift_right_logical(i, 5)
                col = (i & 31) * _L
                v = buf[p, r, pl.ds(col, _L)]
                bits = lax.bitcast_convert_type(v, jnp.int32)
                b = lax.shift_right_logical(bits, shift) & 255
                idx = b * _L + lane
                if shift == 24:
                    plsc.addupdate_scatter(cnt, [idx], ones)
                    return a
                hi = lax.shift_right_logical(bits, shift + 8)
                m = hi == pref_now
                plsc.addupdate_scatter(cnt, [idx], ones, mask=m)
                if shift == 16:
                    m_above = hi > pref_now
                else:
                    m_above = jnp.logical_and(hi > pref_now, hi <= pref_hi)
                av, off = a
                av = av + jnp.where(m_above, v, 0.0)
                if compact:
                    slot = lane_base + jnp.minimum(off, _CAPL - 1)
                    plsc.store_scatter(cbuf, [slot], v, mask=m)
                    off = off + jnp.where(m, 1, 0)
                return (av, off)
            return it_body

        pltpu.async_copy(chunk_src(0), buf.at[0], sema)
        pltpu.async_copy(chunk_src(1), buf.at[1], semb)

        def dchunk_body(m_, acc):
            j0 = 2 * m_
            pltpu.make_async_copy(chunk_src(j0), buf.at[0], sema).wait()
            acc = process(0, acc)
            pltpu.async_copy(chunk_src(j0 + 2), buf.at[0], sema)
            pltpu.make_async_copy(chunk_src(j0 + 1), buf.at[1], semb).wait()
            acc = process(1, acc)
            pltpu.async_copy(chunk_src(j0 + 3), buf.at[1], semb)
            return acc
        if shift == 24:
            sacc = lax.fori_loop(0, _NCHUNK // 2 - 1, dchunk_body, sacc)
        else:
            sacc, offs = lax.fori_loop(
                0, _NCHUNK // 2 - 1, dchunk_body, (sacc, offs))
        pltpu.make_async_copy(chunk_src(_NCHUNK - 2), buf.at[0], sema).wait()
        acc_t = sacc if shift == 24 else (sacc, offs)
        acc_t = process(0, acc_t)
        pltpu.make_async_copy(chunk_src(_NCHUNK - 1), buf.at[1], semb).wait()
        acc_t = process(1, acc_t)
        if shift == 24:
            sacc = acc_t
        else:
            sacc, offs = acc_t

        pltpu.sync_copy(cnt, sh_cnt.at[s])
        rem, s_inc, binstar = merge_and_scan(False, rem, s_inc)
        pref = lax.shift_left(pref, 8) | binstar

    @plsc.parallel_loop(0, _BINS, 1, unroll=8)
    def zero4_body(q):
        cnt[pl.ds(q * _L, _L)] = zeros16
        sm[pl.ds(q * _L, _L)] = zeros16

    pref24 = pref
    maxoff = jnp.max(offs)

    def cbody(i, a):
        slot = lane_base + i
        valid = i < offs
        v = plsc.load_gather(cbuf, [slot], mask=valid)
        bits = lax.bitcast_convert_type(v, jnp.int32)
        hi8 = lax.shift_right_logical(bits, 8)
        m = jnp.logical_and(valid, hi8 == pref24)
        b = bits & 255
        idx = b * _L + lane
        plsc.addupdate_scatter(cnt, [idx], ones, mask=m)
        plsc.addupdate_scatter(sm, [idx], v, mask=m)
        m_above = jnp.logical_and(valid, hi8 > pref24)
        return a + jnp.where(m_above, v, 0.0)
    sacc = lax.fori_loop(0, maxoff, cbody, sacc)

    pltpu.sync_copy(cnt, sh_cnt.at[s])
    pltpu.sync_copy(sm, sh_sum.at[s])
    rem, s_inc, binstar = merge_and_scan(True, rem, s_inc)
    pref = lax.shift_left(pref, 8) | binstar

    plsc.subcore_barrier()
    stage[...] = sacc
    pltpu.sync_copy(stage, sh_mc.at[pl.ds(s * _L, _L)])
    plsc.subcore_barrier()
    pltpu.sync_copy(sh_mc, mc)

    def sred(q, tot):
        return tot + mc[pl.ds(q * _L, _L)]
    sacc_rows = lax.fori_loop(0, _L, sred, zeros16)
    s_above = jnp.sum(sacc_rows)

    tval = lax.bitcast_convert_type(pref, jnp.float32)
    res = s_inc + s_above + rem * tval

    @pl.when(s == 0)
    def _():
        stage[...] = res
        pltpu.sync_copy(stage, out_ref.at[ch])


def _sc_topk_sums(loss):
    mesh = plsc.VectorSubcoreMesh(core_axis_name="c", subcore_axis_name="s")
    f = pl.kernel(
        _sc_body,
        out_type=jax.ShapeDtypeStruct((_C, _L), jnp.float32),
        mesh=mesh,
        compiler_params=pltpu.CompilerParams(
            needs_layout_passes=False, use_tc_tiling_on_sc=True),
        scratch_types=[
            pltpu.VMEM((2, _CH // _W, _W), jnp.float32),
            pltpu.VMEM((_BINS * _L,), jnp.float32),
            pltpu.VMEM((_BINS * _L,), jnp.float32),
            pltpu.VMEM((_L * _CAPL,), jnp.float32),
            pltpu.VMEM((_NS, _BINS), jnp.float32),
            pltpu.VMEM((_BINS,), jnp.float32),
            pltpu.VMEM((_L,), jnp.float32),
            pltpu.VMEM((_BINS,), jnp.float32),
            pltpu.VMEM((_BINS,), jnp.float32),
            pltpu.SemaphoreType.DMA,
            pltpu.SemaphoreType.DMA,
            pltpu.VMEM_SHARED((_NS, _BINS * _L), jnp.float32),
            pltpu.VMEM_SHARED((_NS, _BINS * _L), jnp.float32),
            pltpu.VMEM_SHARED((_BINS,), jnp.float32),
            pltpu.VMEM_SHARED((_BINS,), jnp.float32),
        ],
    )
    return f(loss)


def kernel(predict, target, is_average):
    loss = _bce_loss_channel_major(predict, target)
    sums = _sc_topk_sums(loss)
    total = (sums[0, 0] + sums[1, 0]) / (_KK * _C)
    return jnp.where(is_average, total, total * _B)
